# Initial kernel scaffold; baseline (speedup 1.0000x reference)
import jax
import jax.numpy as jnp
from jax.experimental import pallas as pl


def kernel(token_ids, vocab):
    raise NotImplementedError("write your pallas kernel here")


# speedup vs baseline: 1.5596x; 1.5596x over previous
import functools

import jax
import jax.numpy as jnp
from jax import lax
from jax.experimental import pallas as pl
from jax.experimental.pallas import tpu as pltpu
from jax.experimental.pallas import tpu_sc as plsc

D = 32
NW = 32
CHUNK = 1664


def _make_gather(B):
    b_per_w = B // NW
    n_chunks = b_per_w // CHUNK
    mesh = plsc.VectorSubcoreMesh(core_axis_name="c", subcore_axis_name="s")

    @functools.partial(
        pl.kernel,
        mesh=mesh,
        out_type=jax.ShapeDtypeStruct((B, D), jnp.float32),
        scratch_types=[
            pltpu.VMEM((CHUNK,), jnp.int32),
            pltpu.VMEM((CHUNK, D), jnp.float32),
            pltpu.SemaphoreType.DMA,
        ],
        compiler_params=pltpu.CompilerParams(use_tc_tiling_on_sc=False),
    )
    def k(idx_hbm, table_hbm, out_hbm, idx_v, rows_v, sem):
        wid = lax.axis_index("s") * 2 + lax.axis_index("c")
        base = wid * b_per_w

        def body(i, carry):
            off = base + i * CHUNK
            pltpu.sync_copy(idx_hbm.at[pl.ds(off, CHUNK)], idx_v)
            pltpu.async_copy(table_hbm.at[idx_v], rows_v, sem).wait()
            pltpu.sync_copy(rows_v, out_hbm.at[pl.ds(off, CHUNK)])
            return carry

        lax.fori_loop(0, n_chunks, body, 0)

    return k


def kernel(token_ids, vocab):
    B0, B1 = token_ids.shape
    B = B0 * B1
    flat_idx = token_ids.reshape(B).astype(jnp.int32)
    out = _make_gather(B)(flat_idx, vocab)
    return out.reshape(B0, B1, D)


# speedup vs baseline: 1.5675x; 1.0051x over previous
import functools

import jax
import jax.numpy as jnp
from jax import lax
from jax.experimental import pallas as pl
from jax.experimental.pallas import tpu as pltpu
from jax.experimental.pallas import tpu_sc as plsc

D = 32
NW = 32
CHUNK = 1024
NBUF = 3


def _make_gather(B):
    b_per_w = B // NW
    n_chunks = b_per_w // CHUNK
    mesh = plsc.VectorSubcoreMesh(core_axis_name="c", subcore_axis_name="s")

    @functools.partial(
        pl.kernel,
        mesh=mesh,
        out_type=jax.ShapeDtypeStruct((B, D), jnp.float32),
        scratch_types=[pltpu.VMEM((CHUNK,), jnp.int32)] * NBUF
        + [pltpu.VMEM((CHUNK, D), jnp.float32)] * NBUF
        + [pltpu.SemaphoreType.DMA] * (2 * NBUF),
        compiler_params=pltpu.CompilerParams(use_tc_tiling_on_sc=False),
    )
    def k(idx_hbm, table_hbm, out_hbm, *scratch):
        idx_v = scratch[:NBUF]
        rows_v = scratch[NBUF:2 * NBUF]
        gsem = scratch[2 * NBUF:3 * NBUF]
        osem = scratch[3 * NBUF:]
        wid = lax.axis_index("s") * 2 + lax.axis_index("c")
        base = wid * b_per_w

        gathers = [None] * n_chunks
        stores = [None] * n_chunks

        def issue(i):
            b = i % NBUF
            if i >= NBUF:
                stores[i - NBUF].wait()
            pltpu.sync_copy(idx_hbm.at[pl.ds(base + i * CHUNK, CHUNK)],
                            idx_v[b])
            gathers[i] = pltpu.async_copy(table_hbm.at[idx_v[b]],
                                          rows_v[b], gsem[b])

        def drain(i):
            b = i % NBUF
            gathers[i].wait()
            stores[i] = pltpu.async_copy(
                rows_v[b], out_hbm.at[pl.ds(base + i * CHUNK, CHUNK)],
                osem[b])

        for i in range(min(NBUF - 1, n_chunks)):
            issue(i)
        for i in range(n_chunks):
            if i + NBUF - 1 < n_chunks:
                issue(i + NBUF - 1)
            drain(i)
        for i in range(max(0, n_chunks - NBUF), n_chunks):
            stores[i].wait()

    return k


def kernel(token_ids, vocab):
    B0, B1 = token_ids.shape
    B = B0 * B1
    flat_idx = token_ids.reshape(B).astype(jnp.int32)
    out = _make_gather(B)(flat_idx, vocab)
    return out.reshape(B0, B1, D)
